# Initial kernel scaffold; baseline (speedup 1.0000x reference)
import jax
import jax.numpy as jnp
from jax.experimental import pallas as pl


def kernel(x, params, edge_index, i):
    raise NotImplementedError("write your pallas kernel here")


# speedup vs baseline: 3.7120x; 3.7120x over previous
import functools

import jax
import jax.numpy as jnp
from jax import lax
from jax.experimental import pallas as pl
from jax.experimental.pallas import tpu as pltpu
from jax.experimental.pallas import tpu_sc as plsc

F32 = jnp.float32
N_NODES = 10000
N_EDGES = 320000
D_FEAT = 6
HIDDEN = 64
N_GRAPHS = 32
EPS = 1e-05
BN_EPS = 1e-3

NP = 10240
XW = 16
NC, NS = 2, 16
NW = NC * NS
EB = 128
RW = 80
E_PAD = NW * RW * EB
TR = NP // NS


@functools.cache
def _mesh():
    return plsc.VectorSubcoreMesh(core_axis_name="c", subcore_axis_name="s",
                                  num_cores=NC, num_subcores=NS)


@functools.cache
def _sc_gather_xy_k():
    @functools.partial(
        pl.kernel,
        out_type=(jax.ShapeDtypeStruct((E_PAD, 128), F32),
                  jax.ShapeDtypeStruct((E_PAD, 128), F32)),
        mesh=_mesh(),
        scratch_types=[
            pltpu.VMEM((EB,), jnp.int32),
            pltpu.VMEM((EB, 128), F32),
            pltpu.SemaphoreType.DMA,
        ])
    def _sc_gather_xy(x_hbm, send_hbm, recv_hbm, xs_hbm, xr_hbm,
                      idxv, buf, sem):
        c = lax.axis_index("c")
        s = lax.axis_index("s")
        w = s * NC + c
        e0 = w * RW * EB

        def body(j, carry):
            eo = e0 + j * EB
            pltpu.sync_copy(send_hbm.at[pl.ds(eo, EB)], idxv)
            pltpu.async_copy(x_hbm.at[idxv], buf, sem).wait()
            pltpu.sync_copy(buf, xs_hbm.at[pl.ds(eo, EB)])
            pltpu.sync_copy(recv_hbm.at[pl.ds(eo, EB)], idxv)
            pltpu.async_copy(x_hbm.at[idxv], buf, sem).wait()
            pltpu.sync_copy(buf, xr_hbm.at[pl.ds(eo, EB)])
            return carry

        lax.fori_loop(0, RW, body, 0)
    return _sc_gather_xy


@functools.cache
def _make_sc_scatter(F):
    @functools.partial(
        pl.kernel,
        out_type=jax.ShapeDtypeStruct((NC, NP, F), F32),
        mesh=_mesh(),
        scratch_types=[
            pltpu.VMEM((RW, EB), jnp.int32),
            pltpu.VMEM((EB, F), F32),
            pltpu.VMEM_SHARED((NP, F), F32),
            pltpu.SemaphoreType.DMA,
        ])
    def _k(msgs_hbm, idx_hbm, zeros_hbm, out_hbm, idxv, rows, accum, sem):
        c = lax.axis_index("c")
        s = lax.axis_index("s")
        w = s * NC + c
        r0 = w * RW
        pltpu.sync_copy(zeros_hbm.at[pl.ds(s * TR, TR)],
                        accum.at[pl.ds(s * TR, TR)])
        pltpu.sync_copy(idx_hbm.at[pl.ds(r0, RW)], idxv)
        plsc.subcore_barrier()

        def body(j, carry):
            e0 = (r0 + j) * EB
            pltpu.async_copy(msgs_hbm.at[pl.ds(e0, EB)], rows, sem).wait()
            pltpu.sync_copy(rows, accum.at[idxv.at[j]], add=True)
            return carry

        lax.fori_loop(0, RW, body, 0)
        plsc.subcore_barrier()
        pltpu.sync_copy(accum.at[pl.ds(s * TR, TR)],
                        out_hbm.at[c].at[pl.ds(s * TR, TR)])
    return _k


@functools.cache
def _make_sc_gather_scatter(Kc, F):
    @functools.partial(
        pl.kernel,
        out_type=jax.ShapeDtypeStruct((NC, Kc, NP, F), F32),
        mesh=_mesh(),
        scratch_types=[
            pltpu.VMEM((RW, EB), jnp.int32),
            pltpu.VMEM((RW, EB), jnp.int32),
            pltpu.VMEM((EB, F), F32),
            pltpu.VMEM_SHARED((NP, F), F32),
            pltpu.SemaphoreType.DMA,
        ])
    def _k(y_hbm, sidx_hbm, ridx_hbm, zeros_hbm, out_hbm,
           sidx, ridx, rows, accum, sem):
        c = lax.axis_index("c")
        s = lax.axis_index("s")
        w = s * NC + c
        r0 = w * RW
        pltpu.sync_copy(sidx_hbm.at[pl.ds(r0, RW)], sidx)
        pltpu.sync_copy(ridx_hbm.at[pl.ds(r0, RW)], ridx)
        for k in range(Kc):
            pltpu.sync_copy(zeros_hbm.at[pl.ds(s * TR, TR)],
                            accum.at[pl.ds(s * TR, TR)])
            plsc.subcore_barrier()

            def body(j, carry):
                pltpu.async_copy(y_hbm.at[k].at[ridx.at[j]], rows, sem).wait()
                pltpu.sync_copy(rows, accum.at[sidx.at[j]], add=True)
                return carry

            lax.fori_loop(0, RW, body, 0)
            plsc.subcore_barrier()
            pltpu.sync_copy(accum.at[pl.ds(s * TR, TR)],
                            out_hbm.at[c].at[k].at[pl.ds(s * TR, TR)])
    return _k


_EBLK = 512
_NEB = E_PAD // _EBLK
_NBLK = 1024
_NNB = NP // _NBLK


def _msgs_body(xs_ref, xr_ref, w1p_ref, b1_ref,
               w2_ref, b2_ref, w3_ref, b3_ref, out_ref):
    xs = xs_ref[...]
    xr = xr_ref[...]
    diff = xr - xs
    lane = lax.broadcasted_iota(jnp.int32, (1, 128), 1)
    m03 = (lane < 3).astype(F32)
    m3 = (lane == 3).astype(F32)
    m46 = ((lane >= 4) & (lane < 7)).astype(F32)
    d2 = jnp.sum(diff * diff * m03, axis=1, keepdims=True)
    dist = jnp.sqrt(d2)
    inv = jnp.where(d2 > 0, 1.0 / jnp.where(d2 > 0, dist, 1.0), 0.0)
    vect = diff * inv
    e = (pltpu.roll(diff, 125, 1) * m03 + dist * m3
         + pltpu.roll(vect, 4, 1) * m46)
    h = jnp.maximum(jnp.dot(e, w1p_ref[...], preferred_element_type=F32)
                    + b1_ref[...], 0.0)
    h = jnp.maximum(jnp.dot(h, w2_ref[...], preferred_element_type=F32)
                    + b2_ref[...], 0.0)
    kern = jnp.dot(h, w3_ref[...], preferred_element_type=F32) + b3_ref[...]
    def _bf(v):
        return v.astype(jnp.bfloat16).astype(F32)
    acc = _bf(jnp.broadcast_to(xs[:, 0:1], (xs.shape[0], HIDDEN))) * _bf(kern[:, 0:HIDDEN])
    for f in range(1, D_FEAT):
        acc = acc + (_bf(jnp.broadcast_to(xs[:, f:f + 1], (xs.shape[0], HIDDEN)))
                     * _bf(kern[:, HIDDEN * f:HIDDEN * (f + 1)]))
    out_ref[...] = jnp.concatenate([acc, jnp.zeros_like(acc)], axis=1)


def _tc_msgs(xs, xr, w1p, b1, w2, b2, w3, b3):
    full = lambda shape: pl.BlockSpec(shape, lambda b: (0,) * len(shape))
    return pl.pallas_call(
        _msgs_body,
        grid=(_NEB,),
        in_specs=[
            pl.BlockSpec((_EBLK, 128), lambda b: (b, 0)),
            pl.BlockSpec((_EBLK, 128), lambda b: (b, 0)),
            full((128, HIDDEN)), full((1, HIDDEN)),
            full((HIDDEN, HIDDEN)), full((1, HIDDEN)),
            full((HIDDEN, D_FEAT * HIDDEN)), full((1, D_FEAT * HIDDEN)),
        ],
        out_specs=pl.BlockSpec((_EBLK, 128), lambda b: (b, 0)),
        out_shape=jax.ShapeDtypeStruct((E_PAD, 128), F32),
    )(xs, xr, w1p, b1, w2, b2, w3, b3)


def _ecc_root_body(aggp_ref, x_ref, root_ref, b_ref, w_ref, out_ref):
    agg = (aggp_ref[0] + aggp_ref[1])[:, :HIDDEN]
    xh = jnp.maximum(
        agg + jnp.dot(x_ref[...], root_ref[...], preferred_element_type=F32)
        + b_ref[...], 0.0)
    out_ref[0] = jnp.dot(xh, w_ref[...], preferred_element_type=F32)


def _tc_ecc_root(aggp, x_pad, root_p, ecc_b, g1wp):
    full = lambda shape: pl.BlockSpec(shape, lambda b: (0,) * len(shape))
    return pl.pallas_call(
        _ecc_root_body,
        grid=(_NNB,),
        in_specs=[
            pl.BlockSpec((NC, _NBLK, 128), lambda b: (0, b, 0)),
            pl.BlockSpec((_NBLK, XW), lambda b: (b, 0)),
            full((XW, HIDDEN)), full((1, HIDDEN)), full((HIDDEN, 128)),
        ],
        out_specs=pl.BlockSpec((1, _NBLK, 128), lambda b: (0, b, 0)),
        out_shape=jax.ShapeDtypeStruct((1, NP, 128), F32),
    )(aggp, x_pad, root_p, ecc_b, g1wp)


def _make_tc_layer(Kin, Fin, Fout, Kout):
    def body(z_ref, b_ref, w_ref, out_ref):
        parts = [z_ref[0, k] + z_ref[1, k] for k in range(Kin)]
        z = parts[0] if Kin == 1 else jnp.concatenate(parts, axis=1)
        xh = jnp.maximum(z[:, :Fin] + b_ref[...], 0.0)
        y = jnp.dot(xh, w_ref[...], preferred_element_type=F32)
        for k in range(Kout):
            out_ref[k] = y[:, k * 128:(k + 1) * 128]

    full = lambda shape: pl.BlockSpec(shape, lambda b: (0,) * len(shape))

    def run(zparts, b, w):
        return pl.pallas_call(
            body,
            grid=(_NNB,),
            in_specs=[
                pl.BlockSpec((NC, Kin, _NBLK, 128), lambda b: (0, 0, b, 0)),
                full((1, Fin)), full((Fin, Kout * 128)),
            ],
            out_specs=pl.BlockSpec((Kout, _NBLK, 128),
                                   lambda b: (0, b, 0)),
            out_shape=jax.ShapeDtypeStruct((Kout, NP, 128), F32),
        )(zparts, b, w)
    return run


_tc_l2 = _make_tc_layer(1, 64, 128, 1)
_tc_l3 = _make_tc_layer(1, 128, 256, 2)
_tc_l4 = _make_tc_layer(2, 256, 512, 4)

_PBLK = 1000
_NPB = N_NODES // _PBLK
_F4 = 512


def _pool_body(z_ref, b_ref, icol_ref, ones_ref,
               maxo_ref, sumo_ref, cnto_ref, macc, sacc, cacc):
    b = pl.program_id(0)

    @pl.when(b == 0)
    def _init():
        macc[...] = jnp.full((N_GRAPHS, _F4), -jnp.inf, F32)
        sacc[...] = jnp.zeros((N_GRAPHS, _F4), F32)
        cacc[...] = jnp.zeros((N_GRAPHS, 128), F32)

    parts = [z_ref[0, k] + z_ref[1, k] for k in range(4)]
    xh = jnp.maximum(jnp.concatenate(parts, axis=1) + b_ref[...], 0.0)
    icol = icol_ref[0]
    gid = lax.broadcasted_iota(jnp.int32, (1, N_GRAPHS), 1)
    oh = (icol == gid).astype(F32)
    dn = (((0,), (0,)), ((), ()))
    sacc[...] += lax.dot_general(oh, xh, dn, preferred_element_type=F32,
                                 precision=lax.Precision.HIGHEST)
    cacc[...] += lax.dot_general(oh, ones_ref[...], dn,
                                 preferred_element_type=F32,
                                 precision=lax.Precision.HIGHEST)
    for g in range(N_GRAPHS):
        m = icol == g
        sel = jnp.where(m, xh, -jnp.inf)
        macc[g:g + 1, :] = jnp.maximum(
            macc[g:g + 1, :], jnp.max(sel, axis=0, keepdims=True))

    maxo_ref[...] = macc[...]
    sumo_ref[...] = sacc[...]
    cnto_ref[...] = cacc[...]


def _tc_pool(z4, b4, icol, ones_blk):
    full = lambda shape: pl.BlockSpec(shape, lambda b: (0,) * len(shape))
    return pl.pallas_call(
        _pool_body,
        grid=(_NPB,),
        in_specs=[
            pl.BlockSpec((NC, 4, _PBLK, 128), lambda b: (0, 0, b, 0)),
            full((1, _F4)),
            pl.BlockSpec((1, _PBLK, 1), lambda b: (b, 0, 0)),
            full((_PBLK, 128)),
        ],
        out_specs=[full((N_GRAPHS, _F4)), full((N_GRAPHS, _F4)),
                   full((N_GRAPHS, 128))],
        out_shape=[jax.ShapeDtypeStruct((N_GRAPHS, _F4), F32),
                   jax.ShapeDtypeStruct((N_GRAPHS, _F4), F32),
                   jax.ShapeDtypeStruct((N_GRAPHS, 128), F32)],
        scratch_shapes=[pltpu.VMEM((N_GRAPHS, _F4), F32),
                        pltpu.VMEM((N_GRAPHS, _F4), F32),
                        pltpu.VMEM((N_GRAPHS, 128), F32)],
    )(z4, b4, icol, ones_blk)


def _leaky(v):
    return jnp.where(v >= 0, v, 0.15 * v)


def _head_body(maxp_ref, sump_ref, cnt_ref, *refs):
    (d1w, d1b, g1, be1, mu1, va1,
     d2w, d2b, g2, be2, mu2, va2,
     d3w, d3b, g3, be3, mu3, va3,
     a1w, a1b, a2w, a2b, aow, aob,
     s1w, s1b, s2w, s2b, xu_ref, xs_ref) = refs
    cnt = jnp.maximum(cnt_ref[:, 0:1], 1.0)
    g = jnp.concatenate([maxp_ref[...], sump_ref[...] / cnt], axis=1)
    for (dw, db, gm, bt, mu, va) in ((d1w, d1b, g1, be1, mu1, va1),
                                     (d2w, d2b, g2, be2, mu2, va2),
                                     (d3w, d3b, g3, be3, mu3, va3)):
        g = _leaky(jnp.dot(g, dw[...], preferred_element_type=F32) + db[...])
        g = ((g - mu[...]) * lax.rsqrt(va[...] + BN_EPS) * gm[...] + bt[...])
    xu = jnp.dot(g, a1w[...], preferred_element_type=F32) + a1b[...]
    xu = jnp.dot(xu, a2w[...], preferred_element_type=F32) + a2b[...]
    xu = jnp.dot(xu, aow[...], preferred_element_type=F32) + aob[...]
    n2 = jnp.sum(xu * xu, axis=1, keepdims=True)
    nrm = jnp.sqrt(n2)
    invn = jnp.where(n2 > 0, 1.0 / jnp.where(n2 > 0, nrm, 1.0), 0.0)
    xu_ref[...] = xu * invn
    xs = jnp.dot(g, s1w[...], preferred_element_type=F32) + s1b[...]
    xs = jnp.dot(xs, s2w[...], preferred_element_type=F32) + s2b[...]
    xs_ref[...] = jnp.abs(xs) + EPS


def _tc_head(maxp, sump, cnt, plist):
    full = lambda a: pl.BlockSpec(a.shape, lambda: (0,) * a.ndim)
    args = [maxp, sump, cnt] + plist
    return pl.pallas_call(
        _head_body,
        in_specs=[full(a) for a in args],
        out_specs=[pl.BlockSpec((N_GRAPHS, 128), lambda: (0, 0)),
                   pl.BlockSpec((N_GRAPHS, 128), lambda: (0, 0))],
        out_shape=[jax.ShapeDtypeStruct((N_GRAPHS, 128), F32),
                   jax.ShapeDtypeStruct((N_GRAPHS, 128), F32)],
    )(*args)


def kernel(x, params, edge_index, i):
    p = params
    x_pad = jnp.zeros((NP, XW), F32).at[:N_NODES, :D_FEAT].set(x)
    x_pad128 = jnp.zeros((NP, 128), F32).at[:N_NODES, :D_FEAT].set(x)

    pad_e = E_PAD - N_EDGES
    dummy = N_NODES + (jnp.arange(pad_e, dtype=jnp.int32) % (NP - N_NODES))
    send_p = jnp.concatenate([edge_index[0], dummy])
    recv_p = jnp.concatenate([edge_index[1], dummy])
    send_r = send_p.reshape(NW * RW, EB)
    recv_r = recv_p.reshape(NW * RW, EB)

    w1p = jnp.zeros((128, HIDDEN), F32).at[:7].set(p['kn_W1'])
    b1 = p['kn_b1'][None, :]
    b2 = p['kn_b2'][None, :]
    b3 = p['kn_b3'][None, :]

    root_p = jnp.zeros((XW, HIDDEN), F32).at[:D_FEAT].set(p['ecc_root'])
    zeros64 = jnp.zeros((NP, 64), F32)
    zeros128 = jnp.zeros((NP, 128), F32)

    xs_g, xr_g = _sc_gather_xy_k()(x_pad128, send_p, recv_p)

    msgs = _tc_msgs(xs_g, xr_g, w1p, b1, p['kn_W2'], b2, p['kn_W3'], b3)

    aggp = _make_sc_scatter(128)(msgs, recv_r, zeros128)

    g1wp = jnp.zeros((HIDDEN, 128), F32).at[:, :HIDDEN].set(p['g1_W'])
    y1 = _tc_ecc_root(aggp, x_pad, root_p, p['ecc_b'][None, :], g1wp)

    z1 = _make_sc_gather_scatter(1, 128)(y1, send_r, recv_r, zeros128)
    y2 = _tc_l2(z1, p['g1_b'][None, :], p['g2_W'])
    z2 = _make_sc_gather_scatter(1, 128)(y2, send_r, recv_r, zeros128)
    y3 = _tc_l3(z2, p['g2_b'][None, :], p['g3_W'])
    z3 = _make_sc_gather_scatter(2, 128)(y3, send_r, recv_r, zeros128)
    y4 = _tc_l4(z3, p['g3_b'][None, :], p['g4_W'])
    z4 = _make_sc_gather_scatter(4, 128)(y4, send_r, recv_r, zeros128)

    icol = i.astype(jnp.int32).reshape(_NPB, _PBLK, 1)
    ones_blk = jnp.ones((_PBLK, 128), F32)
    maxp, sump, cnt = _tc_pool(z4, p['g4_b'][None, :], icol, ones_blk)

    aow = jnp.zeros((HIDDEN, 128), F32).at[:, :2].set(p['ao_W'])
    aob = jnp.zeros((1, 128), F32).at[0, :2].set(p['ao_b'])
    s2w = jnp.zeros((HIDDEN, 128), F32).at[:, :HIDDEN].set(p['s2_W'])
    s2b = jnp.zeros((1, 128), F32).at[0, :HIDDEN].set(p['s2_b'])
    plist = [
        p['d1_W'], p['d1_b'][None, :], p['bn1_gamma'][None, :],
        p['bn1_beta'][None, :], p['bn1_mean'][None, :], p['bn1_var'][None, :],
        p['d2_W'], p['d2_b'][None, :], p['bn2_gamma'][None, :],
        p['bn2_beta'][None, :], p['bn2_mean'][None, :], p['bn2_var'][None, :],
        p['d3_W'], p['d3_b'][None, :], p['bn3_gamma'][None, :],
        p['bn3_beta'][None, :], p['bn3_mean'][None, :], p['bn3_var'][None, :],
        p['a1_W'], p['a1_b'][None, :], p['a2_W'], p['a2_b'][None, :],
        aow, aob,
        p['s1_W'], p['s1_b'][None, :], s2w, s2b,
    ]
    xu128, xs128 = _tc_head(maxp, sump, cnt, plist)
    return jnp.concatenate([xu128[:, :2], xs128[:, :HIDDEN]], axis=1)


def _dead_bisect_tail(p, aggp, x, edge_index, i):
    xh = jnp.maximum((aggp[0] + aggp[1])[:N_NODES]
                     + x @ p['ecc_root'] + p['ecc_b'], 0.0)
    send = edge_index[0]
    receive = edge_index[1]
    for li in range(1, 5):
        y = xh @ p[f'g{li}_W']
        xh = jax.nn.relu(jax.ops.segment_sum(y[receive], send,
                                             num_segments=N_NODES) + p[f'g{li}_b'])
    x1 = jax.ops.segment_max(xh, i, num_segments=N_GRAPHS)
    sums = jax.ops.segment_sum(xh, i, num_segments=N_GRAPHS)
    counts = jax.ops.segment_sum(jnp.ones((N_NODES, 1), F32), i, num_segments=N_GRAPHS)
    x2 = sums / jnp.maximum(counts, 1.0)
    g = jnp.concatenate([x1, x2], axis=1)
    for li in range(1, 4):
        t = g @ p[f'd{li}_W'] + p[f'd{li}_b']
        g = jnp.where(t >= 0, t, 0.15 * t)
        g = (g - p[f'bn{li}_mean']) / jnp.sqrt(p[f'bn{li}_var'] + BN_EPS) * p[f'bn{li}_gamma'] + p[f'bn{li}_beta']
    xu = g @ p['a1_W'] + p['a1_b']
    xu = xu @ p['a2_W'] + p['a2_b']
    xu = xu @ p['ao_W'] + p['ao_b']
    nrm = jnp.sqrt(jnp.sum(jnp.square(xu), axis=1))
    safe_n = jnp.where(nrm == 0, 1.0, nrm)
    xu = jnp.where(nrm[:, None] == 0, 0.0, xu / safe_n[:, None])
    xsig = g @ p['s1_W'] + p['s1_b']
    xsig = xsig @ p['s2_W'] + p['s2_b']
    xsig = jnp.abs(xsig) + EPS
    return jnp.concatenate([xu, xsig], axis=1)
